# baseline (device time: 1447337 ns/iter reference)
import jax
import jax.numpy as jnp
from jax import lax
from jax.experimental import pallas as pl
from jax.experimental.pallas import tpu as pltpu

N_DEV = 16
_GELU_C = 0.7978845608028654


def _gelu(y):
    return 0.5 * y * (1.0 + jnp.tanh(_GELU_C * (y + 0.044715 * y * y * y)))


def kernel(x, w_mat):
    m, _ = x.shape
    _, n = w_mat.shape
    m_ch = m // N_DEV

    def body(x_ref, w_ref, out_ref, send_buf, recv_buf,
             send_sem, recv_sems, credit_sem):
        my = lax.axis_index("i")
        left = lax.rem(my + N_DEV - 1, N_DEV)
        right = lax.rem(my + 1, N_DEV)

        barrier = pltpu.get_barrier_semaphore()
        for nbr in (left, right):
            pl.semaphore_signal(
                barrier, inc=1,
                device_id=(nbr,), device_id_type=pl.DeviceIdType.MESH,
            )
        pl.semaphore_wait(barrier, 2)

        def partial_chunk(s):
            c = lax.rem(my + (N_DEV - 1 - s), N_DEV)
            xs = x_ref[pl.ds(c * m_ch, m_ch), :]
            return jnp.dot(xs, w_ref[:, :], preferred_element_type=jnp.float32)

        val = None
        for s in range(N_DEV):
            partial = partial_chunk(s)
            if s == 0:
                val = partial
            else:
                val = partial + recv_buf[s % 2]
            if s < N_DEV - 1:
                if s >= 2:
                    pl.semaphore_wait(credit_sem, 1)
                send_buf[:, :] = val
                rdma = pltpu.make_async_remote_copy(
                    src_ref=send_buf,
                    dst_ref=recv_buf.at[(s + 1) % 2],
                    send_sem=send_sem,
                    recv_sem=recv_sems.at[(s + 1) % 2],
                    device_id=(right,),
                    device_id_type=pl.DeviceIdType.MESH,
                )
                rdma.start()
                rdma.wait()
            if 1 <= s <= N_DEV - 3:
                pl.semaphore_signal(
                    credit_sem, inc=1,
                    device_id=(left,), device_id_type=pl.DeviceIdType.MESH,
                )
        out_ref[:, :] = _gelu(val)

    return pl.pallas_call(
        body,
        out_shape=jax.ShapeDtypeStruct((m_ch, n), jnp.float32),
        in_specs=[
            pl.BlockSpec(memory_space=pltpu.VMEM),
            pl.BlockSpec(memory_space=pltpu.VMEM),
        ],
        out_specs=pl.BlockSpec(memory_space=pltpu.VMEM),
        scratch_shapes=[
            pltpu.VMEM((m_ch, n), jnp.float32),
            pltpu.VMEM((2, m_ch, n), jnp.float32),
            pltpu.SemaphoreType.DMA,
            pltpu.SemaphoreType.DMA((2,)),
            pltpu.SemaphoreType.REGULAR,
        ],
        compiler_params=pltpu.CompilerParams(collective_id=0),
    )(x, w_mat)


# device time: 772945 ns/iter; 1.8725x vs baseline; 1.8725x over previous
import jax
import jax.numpy as jnp
from jax import lax
from jax.experimental import pallas as pl
from jax.experimental.pallas import tpu as pltpu

N_DEV = 16
_GELU_C = 0.7978845608028654


def _gelu(y):
    return 0.5 * y * (1.0 + jnp.tanh(_GELU_C * (y + 0.044715 * y * y * y)))


def kernel(x, w_mat):
    m, _ = x.shape
    _, n = w_mat.shape
    m_ch = m // N_DEV
    n2 = n // 2

    def body(x_ref, w_ref, out_ref,
             send_r, send_l, recv_r, recv_l,
             send_r_sem, send_l_sem, recv_r_sems, recv_l_sems,
             credit_r_sem, credit_l_sem):
        my = lax.axis_index("i")
        left = lax.rem(my + N_DEV - 1, N_DEV)
        right = lax.rem(my + 1, N_DEV)

        barrier = pltpu.get_barrier_semaphore()
        for nbr in (left, right):
            pl.semaphore_signal(
                barrier, inc=1,
                device_id=(nbr,), device_id_type=pl.DeviceIdType.MESH,
            )
        pl.semaphore_wait(barrier, 2)

        def dot_r(s):
            c = lax.rem(my + (N_DEV - 1 - s), N_DEV)
            return jnp.dot(x_ref[pl.ds(c * m_ch, m_ch), :], w_ref[:, :n2],
                           preferred_element_type=jnp.float32)

        def dot_l(s):
            c = lax.rem(my + s + 1, N_DEV)
            return jnp.dot(x_ref[pl.ds(c * m_ch, m_ch), :], w_ref[:, n2:],
                           preferred_element_type=jnp.float32)

        prev_r = prev_l = None
        val_r = val_l = None
        for s in range(N_DEV):
            p_r = dot_r(s)
            p_l = dot_l(s)
            if s == 0:
                val_r, val_l = p_r, p_l
            else:
                prev_r.wait()
                prev_l.wait()
                val_r = p_r + recv_r[s % 2]
                val_l = p_l + recv_l[s % 2]
            if s < N_DEV - 1:
                if s >= 2:
                    pl.semaphore_wait(credit_r_sem, 1)
                    pl.semaphore_wait(credit_l_sem, 1)
                send_r[:, :] = val_r
                send_l[:, :] = val_l
                prev_r = pltpu.make_async_remote_copy(
                    src_ref=send_r,
                    dst_ref=recv_r.at[(s + 1) % 2],
                    send_sem=send_r_sem,
                    recv_sem=recv_r_sems.at[(s + 1) % 2],
                    device_id=(right,),
                    device_id_type=pl.DeviceIdType.MESH,
                )
                prev_l = pltpu.make_async_remote_copy(
                    src_ref=send_l,
                    dst_ref=recv_l.at[(s + 1) % 2],
                    send_sem=send_l_sem,
                    recv_sem=recv_l_sems.at[(s + 1) % 2],
                    device_id=(left,),
                    device_id_type=pl.DeviceIdType.MESH,
                )
                prev_r.start()
                prev_l.start()
            if 1 <= s <= N_DEV - 3:
                pl.semaphore_signal(
                    credit_r_sem, inc=1,
                    device_id=(left,), device_id_type=pl.DeviceIdType.MESH,
                )
                pl.semaphore_signal(
                    credit_l_sem, inc=1,
                    device_id=(right,), device_id_type=pl.DeviceIdType.MESH,
                )
        out_ref[:, :n2] = _gelu(val_r)
        out_ref[:, n2:] = _gelu(val_l)

    return pl.pallas_call(
        body,
        out_shape=jax.ShapeDtypeStruct((m_ch, n), jnp.float32),
        in_specs=[
            pl.BlockSpec(memory_space=pltpu.VMEM),
            pl.BlockSpec(memory_space=pltpu.VMEM),
        ],
        out_specs=pl.BlockSpec(memory_space=pltpu.VMEM),
        scratch_shapes=[
            pltpu.VMEM((m_ch, n2), jnp.float32),
            pltpu.VMEM((m_ch, n2), jnp.float32),
            pltpu.VMEM((2, m_ch, n2), jnp.float32),
            pltpu.VMEM((2, m_ch, n2), jnp.float32),
            pltpu.SemaphoreType.DMA,
            pltpu.SemaphoreType.DMA,
            pltpu.SemaphoreType.DMA((2,)),
            pltpu.SemaphoreType.DMA((2,)),
            pltpu.SemaphoreType.REGULAR,
            pltpu.SemaphoreType.REGULAR,
        ],
        compiler_params=pltpu.CompilerParams(
            collective_id=0,
            vmem_limit_bytes=100 * 1024 * 1024,
        ),
    )(x, w_mat)


# device time: 698366 ns/iter; 2.0725x vs baseline; 1.1068x over previous
import jax
import jax.numpy as jnp
from jax import lax
from jax.experimental import pallas as pl
from jax.experimental.pallas import tpu as pltpu

N_DEV = 16
N_SUB = 2
_GELU_C = 0.7978845608028654


def _gelu(y):
    return 0.5 * y * (1.0 + jnp.tanh(_GELU_C * (y + 0.044715 * y * y * y)))


def kernel(x, w_mat):
    m, _ = x.shape
    _, n = w_mat.shape
    m_ch = m // N_DEV
    n2 = n // 2
    n4 = n2 // N_SUB

    def body(x_ref, w_ref, out_ref,
             send_r, send_l, recv_r, recv_l,
             send_r_sems, send_l_sems, recv_r_sems, recv_l_sems,
             credit_r_sem, credit_l_sem):
        my = lax.axis_index("i")
        left = lax.rem(my + N_DEV - 1, N_DEV)
        right = lax.rem(my + 1, N_DEV)

        barrier = pltpu.get_barrier_semaphore()
        for nbr in (left, right):
            pl.semaphore_signal(
                barrier, inc=1,
                device_id=(nbr,), device_id_type=pl.DeviceIdType.MESH,
            )
        pl.semaphore_wait(barrier, 2)

        def dot_r(s):
            c = lax.rem(my + (N_DEV - 1 - s), N_DEV)
            return jnp.dot(x_ref[pl.ds(c * m_ch, m_ch), :], w_ref[:, :n2],
                           preferred_element_type=jnp.float32)

        def dot_l(s):
            c = lax.rem(my + s + 1, N_DEV)
            return jnp.dot(x_ref[pl.ds(c * m_ch, m_ch), :], w_ref[:, n2:],
                           preferred_element_type=jnp.float32)

        prev = [[None] * N_SUB, [None] * N_SUB]
        for s in range(N_DEV):
            p = (dot_r(s), dot_l(s))
            if 2 <= s < N_DEV - 1:
                pl.semaphore_wait(credit_r_sem, 1)
                pl.semaphore_wait(credit_l_sem, 1)
            for j in range(N_SUB):
                for ring, (send_buf, send_sems, recv_buf, recv_sems, nbr) in (
                    (0, (send_r, send_r_sems, recv_r, recv_r_sems, right)),
                    (1, (send_l, send_l_sems, recv_l, recv_l_sems, left)),
                ):
                    p_sub = p[ring][:, j * n4:(j + 1) * n4]
                    if s == 0:
                        val = p_sub
                    else:
                        prev[ring][j].wait()
                        val = p_sub + recv_buf[s % 2, j]
                    if s < N_DEV - 1:
                        send_buf[j, :, :] = val
                        rdma = pltpu.make_async_remote_copy(
                            src_ref=send_buf.at[j],
                            dst_ref=recv_buf.at[(s + 1) % 2, j],
                            send_sem=send_sems.at[j],
                            recv_sem=recv_sems.at[(s + 1) % 2, j],
                            device_id=(nbr,),
                            device_id_type=pl.DeviceIdType.MESH,
                        )
                        rdma.start()
                        prev[ring][j] = rdma
                    else:
                        col0 = ring * n2 + j * n4
                        out_ref[:, col0:col0 + n4] = _gelu(val)
            if 1 <= s <= N_DEV - 3:
                pl.semaphore_signal(
                    credit_r_sem, inc=1,
                    device_id=(left,), device_id_type=pl.DeviceIdType.MESH,
                )
                pl.semaphore_signal(
                    credit_l_sem, inc=1,
                    device_id=(right,), device_id_type=pl.DeviceIdType.MESH,
                )

    return pl.pallas_call(
        body,
        out_shape=jax.ShapeDtypeStruct((m_ch, n), jnp.float32),
        in_specs=[
            pl.BlockSpec(memory_space=pltpu.VMEM),
            pl.BlockSpec(memory_space=pltpu.VMEM),
        ],
        out_specs=pl.BlockSpec(memory_space=pltpu.VMEM),
        scratch_shapes=[
            pltpu.VMEM((N_SUB, m_ch, n4), jnp.float32),
            pltpu.VMEM((N_SUB, m_ch, n4), jnp.float32),
            pltpu.VMEM((2, N_SUB, m_ch, n4), jnp.float32),
            pltpu.VMEM((2, N_SUB, m_ch, n4), jnp.float32),
            pltpu.SemaphoreType.DMA((N_SUB,)),
            pltpu.SemaphoreType.DMA((N_SUB,)),
            pltpu.SemaphoreType.DMA((2, N_SUB)),
            pltpu.SemaphoreType.DMA((2, N_SUB)),
            pltpu.SemaphoreType.REGULAR,
            pltpu.SemaphoreType.REGULAR,
        ],
        compiler_params=pltpu.CompilerParams(
            collective_id=0,
            vmem_limit_bytes=100 * 1024 * 1024,
        ),
    )(x, w_mat)
